# baseline (speedup 1.0000x reference)
import jax
import jax.numpy as jnp
import numpy as np
from jax import lax
from jax.experimental import pallas as pl
from jax.experimental.pallas import tpu as pltpu
from jax.experimental.pallas import tpu_sc as plsc

NN = 10000
NP = 10240
FF = 128
FH = 64
EE = 320000
EP = 327680
EPT = EP // 16
CHUNK = 128
NCHUNK = EPT // CHUNK
RPT = NP // 16

def _tc_proj_body(x_ref, wl_ref, wr_ref, b_ref, p_ref, q_ref):
    xb = x_ref[...]
    p_ref[...] = jnp.dot(xb, wl_ref[...], preferred_element_type=jnp.float32)
    q_ref[...] = (
        jnp.dot(xb, wr_ref[...], preferred_element_type=jnp.float32)
        + b_ref[...]
    )


def _tc_proj(xp, wl, wr, b):
    blk = 1024
    grid = NP // blk
    rspec = pl.BlockSpec((blk, FF), lambda i: (i, 0))
    wspec = pl.BlockSpec((FF, FF), lambda i: (0, 0))
    return pl.pallas_call(
        _tc_proj_body,
        grid=(grid,),
        in_specs=[rspec, wspec, wspec,
                  pl.BlockSpec((1, FF), lambda i: (0, 0))],
        out_specs=[rspec, rspec],
        out_shape=[
            jax.ShapeDtypeStruct((NP, FF), jnp.float32),
            jax.ShapeDtypeStruct((NP, FF), jnp.float32),
        ],
    )(xp, wl, wr, b.reshape(1, FF))


def _tc_comb_proj_body(sa_ref, sb_ref, p_ref, q_ref, d_ref,
                       wl_ref, wr_ref, b_ref, p2_ref, q2_ref, inv_ref):
    inv = 1.0 / (1.0 + d_ref[...][:, :1])
    inv_ref[...] = inv
    s = jnp.concatenate([sa_ref[...], sb_ref[...]], axis=1)
    h = jnp.maximum((s + p_ref[...]) * inv + q_ref[...], 0.0)
    p2_ref[...] = jnp.dot(h, wl_ref[...], preferred_element_type=jnp.float32)
    q2_ref[...] = (
        jnp.dot(h, wr_ref[...], preferred_element_type=jnp.float32)
        + b_ref[...]
    )


def _tc_comb_proj(sa, sb, p, q, deg, wl, wr, b):
    blk = 1024
    grid = NP // blk
    rspec = pl.BlockSpec((blk, FF), lambda i: (i, 0))
    hspec = pl.BlockSpec((blk, FH), lambda i: (i, 0))
    dspec = pl.BlockSpec((blk, 16), lambda i: (i, 0))
    wspec = pl.BlockSpec((FF, FF), lambda i: (0, 0))
    return pl.pallas_call(
        _tc_comb_proj_body,
        grid=(grid,),
        in_specs=[hspec, hspec, rspec, rspec, dspec, wspec, wspec,
                  pl.BlockSpec((1, FF), lambda i: (0, 0))],
        out_specs=[rspec, rspec,
                   pl.BlockSpec((blk, 1), lambda i: (i, 0))],
        out_shape=[
            jax.ShapeDtypeStruct((NP, FF), jnp.float32),
            jax.ShapeDtypeStruct((NP, FF), jnp.float32),
            jax.ShapeDtypeStruct((NP, 1), jnp.float32),
        ],
    )(sa, sb, p, q, deg, wl, wr, b.reshape(1, FF))


def _tc_comb_body(sa_ref, sb_ref, p_ref, q_ref, inv_ref, o_ref):
    s = jnp.concatenate([sa_ref[...], sb_ref[...]], axis=1)
    o_ref[...] = jnp.maximum(
        (s + p_ref[...]) * inv_ref[...] + q_ref[...], 0.0)


def _tc_comb(sa, sb, p, q, inv):
    blk = 1024
    grid = NP // blk
    rspec = pl.BlockSpec((blk, FF), lambda i: (i, 0))
    hspec = pl.BlockSpec((blk, FH), lambda i: (i, 0))
    ispec = pl.BlockSpec((blk, 1), lambda i: (i, 0))
    return pl.pallas_call(
        _tc_comb_body,
        grid=(grid,),
        in_specs=[hspec, hspec, rspec, rspec, ispec],
        out_specs=rspec,
        out_shape=jax.ShapeDtypeStruct((NP, FF), jnp.float32),
    )(sa, sb, p, q, inv)


def _make_sc_scatter(compute_deg: bool, nbuf: int):
    mesh = plsc.VectorSubcoreMesh(core_axis_name="c", subcore_axis_name="s")
    out_type = [jax.ShapeDtypeStruct((2, NP, FH), jnp.float32)]
    if compute_deg:
        out_type.append(jax.ShapeDtypeStruct((NP, 16), jnp.float32))
    scratch = []
    for _ in range(nbuf):
        scratch += [
            pltpu.VMEM((CHUNK, FH), jnp.float32),
            pltpu.SemaphoreType.DMA,
            pltpu.SemaphoreType.DMA,
        ]
    scratch += [
        pltpu.VMEM((NCHUNK, 128), jnp.int32),
        pltpu.VMEM((NCHUNK, 128), jnp.int32),
        pltpu.SemaphoreType.DMA,
        pltpu.VMEM((16, FH), jnp.float32),
        pltpu.VMEM_SHARED((NP, FH), jnp.float32),
    ]
    if compute_deg:
        scratch += [
            pltpu.VMEM((128, 16), jnp.float32),
            pltpu.VMEM((64, 16), jnp.float32),
            pltpu.VMEM_SHARED((NP, 16), jnp.float32),
        ]

    def body(pa_hbm, pb_hbm, src_hbm, dst_hbm, *refs):
        if compute_deg:
            (sacc_hbm, deg_hbm, *dmarefs, srcpre, dstpre, semi, zbuf, acc,
             onesb, zb16, dacc) = refs
        else:
            (sacc_hbm, *dmarefs, srcpre, dstpre, semi, zbuf, acc) = refs
        bufs = [tuple(dmarefs[3 * k:3 * k + 3]) for k in range(nbuf)]
        c = lax.axis_index("c")
        s = lax.axis_index("s")

        pltpu.async_copy(
            src_hbm.at[pl.ds(s * NCHUNK, NCHUNK)], srcpre, semi)
        pltpu.async_copy(
            dst_hbm.at[pl.ds(s * NCHUNK, NCHUNK)], dstpre, semi)

        z16 = jnp.zeros((16,), jnp.float32)
        for r in range(16):
            for k in range(FH // 16):
                zbuf[r, pl.ds(k * 16, 16)] = z16

        def zero_acc(i, _):
            pltpu.sync_copy(zbuf, acc.at[pl.ds(s * RPT + i * 16, 16)])
            return 0
        lax.fori_loop(0, RPT // 16, zero_acc, 0)

        if compute_deg:
            o16 = jnp.ones((16,), jnp.float32)

            def fill_ones(i, _):
                onesb[i, pl.ds(0, 16)] = o16
                return 0
            lax.fori_loop(0, 128, fill_ones, 0)

            def zero_zb16(i, _):
                zb16[i, pl.ds(0, 16)] = z16
                return 0
            lax.fori_loop(0, 64, zero_zb16, 0)

            @pl.when(c == 0)
            def _():
                def zero_dacc(i, _):
                    pltpu.sync_copy(
                        zb16, dacc.at[pl.ds(s * RPT + i * 64, 64)])
                    return 0
                lax.fori_loop(0, RPT // 64, zero_dacc, 0)

        plsc.subcore_barrier()

        pltpu.make_async_copy(
            src_hbm.at[pl.ds(s * NCHUNK, NCHUNK)], srcpre, semi).wait()
        pltpu.make_async_copy(
            dst_hbm.at[pl.ds(s * NCHUNK, NCHUNK)], dstpre, semi).wait()

        def run(p_hbm, do_deg):
            def issue_g(buf, g):
                pltpu.async_copy(p_hbm.at[srcpre.at[g]], buf[0], buf[1])

            def wait_g(buf, g):
                pltpu.make_async_copy(
                    p_hbm.at[srcpre.at[g]], buf[0], buf[1]).wait()

            def issue_s(buf, g):
                pltpu.async_copy(
                    buf[0], acc.at[dstpre.at[g]], buf[2], add=True)
                if do_deg:
                    pltpu.async_copy(
                        onesb, dacc.at[dstpre.at[g]], buf[2], add=True)

            def wait_s(buf, g):
                pltpu.make_async_copy(
                    buf[0], acc.at[dstpre.at[g]], buf[2]).wait()
                if do_deg:
                    pltpu.make_async_copy(
                        onesb, dacc.at[dstpre.at[g]], buf[2]).wait()

            def chunk(r, cc, wait_prev=True, guard=False):
                cur = bufs[r % nbuf]
                prev = bufs[(r - 1) % nbuf]
                wait_g(cur, cc)
                issue_s(cur, cc)
                if wait_prev:
                    wait_s(prev, cc - 1)
                pf = cc + nbuf - 1
                if guard:
                    @pl.when(pf <= NCHUNK - 1)
                    def _():
                        issue_g(prev, pf)
                else:
                    issue_g(prev, pf)

            for k in range(nbuf - 1):
                issue_g(bufs[k], k)
            chunk(0, 0, wait_prev=False)
            for r in range(1, nbuf):
                chunk(r, r)

            def step(u, _):
                cbase = nbuf * u
                for r in range(nbuf):
                    chunk(r, cbase + r, guard=True)
                return 0
            lax.fori_loop(1, NCHUNK // nbuf, step, 0)

            wait_s(bufs[(NCHUNK - 1) % nbuf], NCHUNK - 1)

        @pl.when(c == 0)
        def _():
            run(pa_hbm, compute_deg)

        @pl.when(c == 1)
        def _():
            run(pb_hbm, False)

        plsc.subcore_barrier()

        pltpu.sync_copy(
            acc.at[pl.ds(s * RPT, RPT)],
            sacc_hbm.at[c, pl.ds(s * RPT, RPT)])

        if compute_deg:
            @pl.when(c == 0)
            def _():
                pltpu.sync_copy(
                    dacc.at[pl.ds(s * RPT, RPT)],
                    deg_hbm.at[pl.ds(s * RPT, RPT)])

    return pl.kernel(
        body, mesh=mesh, out_type=out_type, scratch_types=scratch,
        compiler_params=pltpu.CompilerParams(use_tc_tiling_on_sc=False))


_sc_scatter_deg = _make_sc_scatter(True, 4)
_sc_scatter = _make_sc_scatter(False, 5)


@jax.jit
def kernel(x, edge_index, W1l, W1r, b1, W2l, W2r, b2):
    xp = jnp.zeros((NP, FF), jnp.float32).at[:NN].set(x)
    pad = EP - EE
    srcp = jnp.concatenate(
        [edge_index[0], jnp.zeros((pad,), jnp.int32)]).reshape(EP // 128, 128)
    dstp = jnp.concatenate(
        [edge_index[1], jnp.full((pad,), NN, jnp.int32)]).reshape(EP // 128, 128)
    p1, q1 = _tc_proj(xp, W1l, W1r, b1)
    sacc1, deg = _sc_scatter_deg(p1[:, :FH], p1[:, FH:], srcp, dstp)
    p2, q2, inv = _tc_comb_proj(sacc1[0], sacc1[1], p1, q1,
                                deg, W2l, W2r, b2)
    (sacc2,) = _sc_scatter(p2[:, :FH], p2[:, FH:], srcp, dstp)
    out = _tc_comb(sacc2[0], sacc2[1], p2, q2, inv)
    return out[:NN]
